# baseline (speedup 1.0000x reference)
import functools

import jax
import jax.numpy as jnp
import numpy as np
from jax import lax
from jax.experimental import pallas as pl
from jax.experimental.pallas import tpu as pltpu
from jax.experimental.pallas import tpu_sc as plsc

B, N, NMASKS, SEEDS, K, D = 8, 8192, 10, 256, 64, 64
R = 0.05
SB = 256
NSB = SEEDS // SB
FC = 128
MPAD = 16
BITS_3F = int(np.float32(3.0).view(np.int32))
I32_MAX = np.int32(2**31 - 1)


def _radius_d2_bound():
    r = np.float32(R)
    t = r * r
    up = np.float32(np.inf)
    dn = np.float32(-np.inf)
    while np.sqrt(np.nextafter(t, up)) <= r:
        t = np.nextafter(t, up)
    while np.sqrt(t) > r:
        t = np.nextafter(t, dn)
    return float(t)


R2BOUND = _radius_d2_bound()


def _fps_body(post_ref, out_ref):
    px = post_ref[0]
    py = post_ref[1]
    pz = post_ref[2]
    lane = lax.broadcasted_iota(jnp.int32, (B, N), 1).astype(jnp.float32)
    slane = lax.broadcasted_iota(jnp.int32, (B, SEEDS), 1)

    lx0 = px[:, 0:1]
    ly0 = py[:, 0:1]
    lz0 = pz[:, 0:1]
    zeros_s = jnp.zeros((B, SEEDS), jnp.float32)
    w0 = slane == 0
    sxs = jnp.where(w0, lx0, zeros_s)
    sys_ = jnp.where(w0, ly0, zeros_s)
    szs = jnp.where(w0, lz0, zeros_s)
    idxs = zeros_s
    dists = jnp.full((B, N), 1e10, jnp.float32)

    off = jnp.float32(8388608.0)
    laneoff = lane + off

    def body(i, carry):
        dists, idxs, sxs, sys_, szs, lx, ly, lz = carry
        dx = px - lx
        dy = py - ly
        dz = pz - lz
        d = (dx * dx + dy * dy) + dz * dz
        dists = jnp.minimum(dists, d)
        m = jnp.max(dists, axis=1, keepdims=True)
        hotf = (dists == m).astype(jnp.float32)
        si = jnp.sum(hotf * laneoff, axis=1, keepdims=True)
        ex = jnp.sum(hotf * px, axis=1, keepdims=True)
        ey = jnp.sum(hotf * py, axis=1, keepdims=True)
        ez = jnp.sum(hotf * pz, axis=1, keepdims=True)

        def tie_path(_):
            hot = dists == m
            nxt = jnp.min(jnp.where(hot, lane, 1e9), axis=1, keepdims=True)
            h2 = lane == nxt
            tx = jnp.sum(jnp.where(h2, px, 0.0), axis=1, keepdims=True)
            ty = jnp.sum(jnp.where(h2, py, 0.0), axis=1, keepdims=True)
            tz = jnp.sum(jnp.where(h2, pz, 0.0), axis=1, keepdims=True)
            return nxt, tx, ty, tz

        def fast_path(_):
            return si - off, ex, ey, ez

        nxt, lx, ly, lz = lax.cond(jnp.max(si) >= 2.0 * off, tie_path,
                                   fast_path, 0)
        w = slane == i
        idxs = jnp.where(w, nxt, idxs)
        sxs = jnp.where(w, lx, sxs)
        sys_ = jnp.where(w, ly, sys_)
        szs = jnp.where(w, lz, szs)
        return dists, idxs, sxs, sys_, szs, lx, ly, lz

    carry = (dists, idxs, sxs, sys_, szs, lx0, ly0, lz0)
    _, idxs, sxs, sys_, szs, _, _, _ = lax.fori_loop(1, SEEDS, body, carry, unroll=15)
    out_ref[:, 0:1, :] = idxs[:, None, :]
    out_ref[:, 1:2, :] = sxs[:, None, :]
    out_ref[:, 2:3, :] = sys_[:, None, :]
    out_ref[:, 3:4, :] = szs[:, None, :]


def _fps(posT):
    return pl.pallas_call(
        _fps_body,
        out_shape=jax.ShapeDtypeStruct((B, 4, SEEDS), jnp.float32),
    )(posT)


_NC, _NS = 2, 16
_NW = _NC * _NS
_RPW = (B * SEEDS) // _NW


def _sc_gather_body(idxf_hbm, table_hbm, out_hbm, idxf_v, idx_v, rows_v, sem):
    wid = lax.axis_index("s") * _NC + lax.axis_index("c")
    base = wid * _RPW
    boff = (base // SEEDS) * N
    pltpu.sync_copy(idxf_hbm.at[pl.ds(base, _RPW)], idxf_v)
    for j in range(_RPW // 16):
        idx_v[pl.ds(j * 16, 16)] = idxf_v[pl.ds(j * 16, 16)].astype(jnp.int32) + boff
    pltpu.async_copy(table_hbm.at[idx_v], rows_v, sem).wait()
    pltpu.sync_copy(rows_v, out_hbm.at[pl.ds(base, _RPW)])


def _sc_gather(idxf, table):
    mesh = plsc.VectorSubcoreMesh(core_axis_name="c", subcore_axis_name="s")
    fn = functools.partial(
        pl.kernel,
        mesh=mesh,
        out_type=jax.ShapeDtypeStruct((B * SEEDS, MPAD), jnp.float32),
        scratch_types=[
            pltpu.VMEM((_RPW,), jnp.float32),
            pltpu.VMEM((_RPW,), jnp.int32),
            pltpu.VMEM((_RPW, MPAD), jnp.float32),
            pltpu.SemaphoreType.DMA,
        ],
        compiler_params=pltpu.CompilerParams(use_tc_tiling_on_sc=False),
    )(_sc_gather_body)
    return fn(idxf, table)


def _group_body(pos_ref, masks_ref, post_ref, seed_ref,
                w1_ref, b1_ref, w2_ref, b2_ref, wg_ref, bg_ref,
                wc1_ref, bc1_ref, wc2_ref, bc2_ref, out_ref, f_scr):
    sb = pl.program_id(1)

    @pl.when(sb == 0)
    def _build_features():
        p = pos_ref[0]
        h = jnp.maximum(
            jnp.dot(p, w1_ref[...], preferred_element_type=jnp.float32)
            + b1_ref[...], 0.0)
        x = jnp.maximum(
            jnp.dot(h, w2_ref[...], preferred_element_type=jnp.float32)
            + b2_ref[...], 0.0)
        ones = jnp.ones((N, 1), jnp.float32)
        zeros = jnp.zeros((N, FC - D - 1), jnp.float32)
        f_scr[...] = jnp.concatenate([x, ones, zeros],
                                     axis=1).astype(jnp.bfloat16)

    info = seed_ref[0, :, pl.ds(sb * SB, SB)]
    infoT = jnp.swapaxes(info, 0, 1)
    idxc = infoT[:, 0:1]
    sx = infoT[:, 1:2]
    sy = infoT[:, 2:3]
    sz = infoT[:, 3:4]

    px = post_ref[0, 0:1, :]
    py = post_ref[0, 1:2, :]
    pz = post_ref[0, 2:3, :]
    dx = sx - px
    dy = sy - py
    dz = sz - pz
    d2 = (dx * dx + dy * dy) + dz * dz
    maskR = (d2 <= R2BOUND).astype(jnp.bfloat16)

    lanei = lax.broadcasted_iota(jnp.int32, (SB, N), 1)
    maskH = (lanei == idxc.astype(jnp.int32)).astype(jnp.bfloat16)

    f = f_scr[...]
    aggR = jnp.dot(maskR, f, preferred_element_type=jnp.float32)
    aggH = jnp.dot(maskH, f, preferred_element_type=jnp.float32)

    kf = jnp.float32(K)
    cntR = aggR[:, D:D + 1]

    def refine(_):
        d2b = lax.bitcast_convert_type(d2, jnp.int32)
        lo = jnp.zeros((SB, 1), jnp.int32)
        hi = jnp.full((SB, 1), BITS_3F, jnp.int32)

        def bs(_, lh):
            lo, hi = lh
            mid = (lo + hi) // 2
            c = jnp.sum((d2b <= mid).astype(jnp.float32), axis=1, keepdims=True)
            ge = c >= kf
            return jnp.where(ge, lo, mid + 1), jnp.where(ge, mid, hi)

        lo, hi = lax.fori_loop(0, 31, bs, (lo, hi))
        tau = jnp.where(cntR > kf, hi, I32_MAX)
        maskN = jnp.minimum(maskR, (d2b <= tau).astype(jnp.bfloat16))
        return jnp.dot(maskN, f, preferred_element_type=jnp.float32)

    aggN = lax.cond(jnp.max(cntR) > kf, refine, lambda _: aggR, 0)

    nn_sum = aggN[:, 0:D]
    cnt = aggN[:, D:D + 1]
    seed_x = aggH[:, 0:D]

    nn_avg = nn_sum / jnp.maximum(cnt, 1e-9)
    disc = seed_x - nn_avg
    gin = jnp.concatenate([seed_x, nn_avg], axis=1)
    grp = jnp.maximum(
        jnp.dot(gin, wg_ref[...], preferred_element_type=jnp.float32)
        + bg_ref[...], 0.0) + disc
    h = jnp.maximum(
        jnp.dot(grp, wc1_ref[...], preferred_element_type=jnp.float32)
        + bc1_ref[...], 0.0)
    out_ref[0] = (jnp.dot(h, wc2_ref[...], preferred_element_type=jnp.float32)
                  + bc2_ref[...])


def _group(pos, masks, posT, seedinfo, W1, b1, W2, b2,
           Wg, bg, Wc1, bc1, Wc2, bc2):
    return pl.pallas_call(
        _group_body,
        grid=(B, NSB),
        in_specs=[
            pl.BlockSpec((1, N, 3), lambda b, s: (b, 0, 0)),
            pl.BlockSpec((1, N, NMASKS), lambda b, s: (b, 0, 0)),
            pl.BlockSpec((1, 3, N), lambda b, s: (b, 0, 0)),
            pl.BlockSpec((1, 4, SEEDS), lambda b, s: (b, 0, 0)),
            pl.BlockSpec((3, D), lambda b, s: (0, 0)),
            pl.BlockSpec((1, D), lambda b, s: (0, 0)),
            pl.BlockSpec((D, D), lambda b, s: (0, 0)),
            pl.BlockSpec((1, D), lambda b, s: (0, 0)),
            pl.BlockSpec((2 * D, D), lambda b, s: (0, 0)),
            pl.BlockSpec((1, D), lambda b, s: (0, 0)),
            pl.BlockSpec((D, D // 2), lambda b, s: (0, 0)),
            pl.BlockSpec((1, D // 2), lambda b, s: (0, 0)),
            pl.BlockSpec((D // 2, NMASKS), lambda b, s: (0, 0)),
            pl.BlockSpec((1, NMASKS), lambda b, s: (0, 0)),
        ],
        out_specs=pl.BlockSpec((1, SB, NMASKS), lambda b, s: (b, s, 0)),
        out_shape=jax.ShapeDtypeStruct((B, SEEDS, NMASKS), jnp.float32),
        scratch_shapes=[pltpu.VMEM((N, FC), jnp.bfloat16)],
        compiler_params=pltpu.CompilerParams(
            dimension_semantics=("arbitrary", "arbitrary")),
    )(pos, masks, posT, seedinfo, W1, b1.reshape(1, D), W2,
      b2.reshape(1, D), Wg, bg.reshape(1, D), Wc1, bc1.reshape(1, D // 2),
      Wc2, bc2.reshape(1, NMASKS))


def _add_body(lg_ref, sm_ref, out_ref):
    out_ref[...] = lg_ref[...] + sm_ref[:, :, 0:NMASKS]


def _addmasks(logits0, smasks):
    return pl.pallas_call(
        _add_body,
        out_shape=jax.ShapeDtypeStruct((B, SEEDS, NMASKS), jnp.float32),
    )(logits0, smasks)


@jax.jit
def kernel(pos, masks, W1, b1, W2, b2, Wg, bg, Wc1, bc1, Wc2, bc2):
    pos = pos.astype(jnp.float32)
    masks = masks.astype(jnp.float32)
    posT = jnp.transpose(pos, (0, 2, 1))
    seedinfo = _fps(jnp.transpose(pos, (2, 0, 1)))
    table = jnp.pad(masks.reshape(B * N, NMASKS),
                    ((0, 0), (0, MPAD - NMASKS)))
    smasks = _sc_gather(seedinfo[:, 0, :].reshape(B * SEEDS), table)
    logits0 = _group(pos, masks, posT, seedinfo, W1, b1, W2, b2,
                     Wg, bg, Wc1, bc1, Wc2, bc2)
    return _addmasks(logits0, smasks.reshape(B, SEEDS, MPAD))


# speedup vs baseline: 1.0356x; 1.0356x over previous
import functools

import jax
import jax.numpy as jnp
import numpy as np
from jax import lax
from jax.experimental import pallas as pl
from jax.experimental.pallas import tpu as pltpu
from jax.experimental.pallas import tpu_sc as plsc

B, N, NMASKS, SEEDS, K, D = 8, 8192, 10, 256, 64, 64
R = 0.05
SB = 256
NSB = SEEDS // SB
FC = 128
MPAD = 16
BITS_3F = int(np.float32(3.0).view(np.int32))
I32_MAX = np.int32(2**31 - 1)


def _radius_d2_bound():
    r = np.float32(R)
    t = r * r
    up = np.float32(np.inf)
    dn = np.float32(-np.inf)
    while np.sqrt(np.nextafter(t, up)) <= r:
        t = np.nextafter(t, up)
    while np.sqrt(t) > r:
        t = np.nextafter(t, dn)
    return float(t)


R2BOUND = _radius_d2_bound()


def _fps_body(post_ref, out_ref):
    px = post_ref[0]
    py = post_ref[1]
    pz = post_ref[2]
    lane = lax.broadcasted_iota(jnp.int32, (B, N), 1).astype(jnp.float32)
    slane = lax.broadcasted_iota(jnp.int32, (B, SEEDS), 1)

    lx0 = px[:, 0:1]
    ly0 = py[:, 0:1]
    lz0 = pz[:, 0:1]
    zeros_s = jnp.zeros((B, SEEDS), jnp.float32)
    w0 = slane == 0
    sxs = jnp.where(w0, lx0, zeros_s)
    sys_ = jnp.where(w0, ly0, zeros_s)
    szs = jnp.where(w0, lz0, zeros_s)
    idxs = zeros_s
    dists = jnp.full((B, N), 1e10, jnp.float32)

    off = jnp.float32(8388608.0)
    laneoff = lane + off

    def body(i, carry):
        dists, idxs, sxs, sys_, szs, lx, ly, lz = carry
        dx = px - lx
        dy = py - ly
        dz = pz - lz
        d = (dx * dx + dy * dy) + dz * dz
        dists = jnp.minimum(dists, d)
        m = jnp.max(dists, axis=1, keepdims=True)
        hotf = (dists == m).astype(jnp.float32)
        si = jnp.sum(hotf * laneoff, axis=1, keepdims=True)
        ex = jnp.sum(hotf * px, axis=1, keepdims=True)
        ey = jnp.sum(hotf * py, axis=1, keepdims=True)
        ez = jnp.sum(hotf * pz, axis=1, keepdims=True)

        def tie_path(_):
            hot = dists == m
            nxt = jnp.min(jnp.where(hot, lane, 1e9), axis=1, keepdims=True)
            h2 = lane == nxt
            tx = jnp.sum(jnp.where(h2, px, 0.0), axis=1, keepdims=True)
            ty = jnp.sum(jnp.where(h2, py, 0.0), axis=1, keepdims=True)
            tz = jnp.sum(jnp.where(h2, pz, 0.0), axis=1, keepdims=True)
            return nxt, tx, ty, tz

        def fast_path(_):
            return si - off, ex, ey, ez

        nxt, lx, ly, lz = lax.cond(jnp.max(si) >= 2.0 * off, tie_path,
                                   fast_path, 0)
        w = slane == i
        idxs = jnp.where(w, nxt, idxs)
        sxs = jnp.where(w, lx, sxs)
        sys_ = jnp.where(w, ly, sys_)
        szs = jnp.where(w, lz, szs)
        return dists, idxs, sxs, sys_, szs, lx, ly, lz

    carry = (dists, idxs, sxs, sys_, szs, lx0, ly0, lz0)
    _, idxs, sxs, sys_, szs, _, _, _ = lax.fori_loop(1, SEEDS, body, carry, unroll=15)
    out_ref[:, 0:1, :] = idxs[:, None, :]
    out_ref[:, 1:2, :] = sxs[:, None, :]
    out_ref[:, 2:3, :] = sys_[:, None, :]
    out_ref[:, 3:4, :] = szs[:, None, :]


def _fps(posT):
    return pl.pallas_call(
        _fps_body,
        out_shape=jax.ShapeDtypeStruct((B, 4, SEEDS), jnp.float32),
    )(posT)


_NC, _NS = 2, 16
_NW = _NC * _NS
_RPW = (B * SEEDS) // _NW


def _sc_gather_body(idxf_hbm, table_hbm, out_hbm, idxf_v, idx_v, rows_v, sem):
    wid = lax.axis_index("s") * _NC + lax.axis_index("c")
    base = wid * _RPW
    boff = (base // SEEDS) * N
    pltpu.sync_copy(idxf_hbm.at[pl.ds(base, _RPW)], idxf_v)
    for j in range(_RPW // 16):
        idx_v[pl.ds(j * 16, 16)] = idxf_v[pl.ds(j * 16, 16)].astype(jnp.int32) + boff
    pltpu.async_copy(table_hbm.at[idx_v], rows_v, sem).wait()
    pltpu.sync_copy(rows_v, out_hbm.at[pl.ds(base, _RPW)])


def _sc_gather(idxf, table):
    mesh = plsc.VectorSubcoreMesh(core_axis_name="c", subcore_axis_name="s")
    fn = functools.partial(
        pl.kernel,
        mesh=mesh,
        out_type=jax.ShapeDtypeStruct((B * SEEDS, MPAD), jnp.float32),
        scratch_types=[
            pltpu.VMEM((_RPW,), jnp.float32),
            pltpu.VMEM((_RPW,), jnp.int32),
            pltpu.VMEM((_RPW, MPAD), jnp.float32),
            pltpu.SemaphoreType.DMA,
        ],
        compiler_params=pltpu.CompilerParams(use_tc_tiling_on_sc=False),
    )(_sc_gather_body)
    return fn(idxf, table)


def _group_body(pos_ref, masks_ref, post_ref, seed_ref,
                w1_ref, b1_ref, w2_ref, b2_ref, wg_ref, bg_ref,
                wc1_ref, bc1_ref, wc2_ref, bc2_ref, out_ref, f_scr):
    sb = pl.program_id(1)

    @pl.when(sb == 0)
    def _build_features():
        p = pos_ref[0]
        h = jnp.maximum(
            jnp.dot(p, w1_ref[...], preferred_element_type=jnp.float32)
            + b1_ref[...], 0.0)
        x = jnp.maximum(
            jnp.dot(h, w2_ref[...], preferred_element_type=jnp.float32)
            + b2_ref[...], 0.0)
        ones = jnp.ones((N, 1), jnp.float32)
        zeros = jnp.zeros((N, FC - D - 1), jnp.float32)
        f_scr[...] = jnp.concatenate([x, ones, zeros], axis=1)

    info = seed_ref[0, :, pl.ds(sb * SB, SB)]
    infoT = jnp.swapaxes(info, 0, 1)
    idxc = infoT[:, 0:1]
    sx = infoT[:, 1:2]
    sy = infoT[:, 2:3]
    sz = infoT[:, 3:4]

    px = post_ref[0, 0:1, :]
    py = post_ref[0, 1:2, :]
    pz = post_ref[0, 2:3, :]
    dx = sx - px
    dy = sy - py
    dz = sz - pz
    d2 = (dx * dx + dy * dy) + dz * dz
    maskR = (d2 <= R2BOUND).astype(jnp.float32)

    lanei = lax.broadcasted_iota(jnp.int32, (SB, N), 1)
    maskH = (lanei == idxc.astype(jnp.int32)).astype(jnp.float32)

    f = f_scr[...]
    aggR = jnp.dot(maskR, f, preferred_element_type=jnp.float32)
    aggH = jnp.dot(maskH, f, preferred_element_type=jnp.float32)

    kf = jnp.float32(K)
    cntR = aggR[:, D:D + 1]

    def refine(_):
        d2b = lax.bitcast_convert_type(d2, jnp.int32)
        lo = jnp.zeros((SB, 1), jnp.int32)
        hi = jnp.full((SB, 1), BITS_3F, jnp.int32)

        def bs(_, lh):
            lo, hi = lh
            mid = (lo + hi) // 2
            c = jnp.sum((d2b <= mid).astype(jnp.float32), axis=1, keepdims=True)
            ge = c >= kf
            return jnp.where(ge, lo, mid + 1), jnp.where(ge, mid, hi)

        lo, hi = lax.fori_loop(0, 31, bs, (lo, hi))
        tau = jnp.where(cntR > kf, hi, I32_MAX)
        maskN = jnp.minimum(maskR, (d2b <= tau).astype(jnp.float32))
        return jnp.dot(maskN, f, preferred_element_type=jnp.float32)

    aggN = lax.cond(jnp.max(cntR) > kf, refine, lambda _: aggR, 0)

    nn_sum = aggN[:, 0:D]
    cnt = aggN[:, D:D + 1]
    seed_x = aggH[:, 0:D]

    nn_avg = nn_sum / jnp.maximum(cnt, 1e-9)
    disc = seed_x - nn_avg
    gin = jnp.concatenate([seed_x, nn_avg], axis=1)
    grp = jnp.maximum(
        jnp.dot(gin, wg_ref[...], preferred_element_type=jnp.float32)
        + bg_ref[...], 0.0) + disc
    h = jnp.maximum(
        jnp.dot(grp, wc1_ref[...], preferred_element_type=jnp.float32)
        + bc1_ref[...], 0.0)
    out_ref[0] = (jnp.dot(h, wc2_ref[...], preferred_element_type=jnp.float32)
                  + bc2_ref[...])


def _group(pos, masks, posT, seedinfo, W1, b1, W2, b2,
           Wg, bg, Wc1, bc1, Wc2, bc2):
    return pl.pallas_call(
        _group_body,
        grid=(B, NSB),
        in_specs=[
            pl.BlockSpec((1, N, 3), lambda b, s: (b, 0, 0)),
            pl.BlockSpec((1, N, NMASKS), lambda b, s: (b, 0, 0)),
            pl.BlockSpec((1, 3, N), lambda b, s: (b, 0, 0)),
            pl.BlockSpec((1, 4, SEEDS), lambda b, s: (b, 0, 0)),
            pl.BlockSpec((3, D), lambda b, s: (0, 0)),
            pl.BlockSpec((1, D), lambda b, s: (0, 0)),
            pl.BlockSpec((D, D), lambda b, s: (0, 0)),
            pl.BlockSpec((1, D), lambda b, s: (0, 0)),
            pl.BlockSpec((2 * D, D), lambda b, s: (0, 0)),
            pl.BlockSpec((1, D), lambda b, s: (0, 0)),
            pl.BlockSpec((D, D // 2), lambda b, s: (0, 0)),
            pl.BlockSpec((1, D // 2), lambda b, s: (0, 0)),
            pl.BlockSpec((D // 2, NMASKS), lambda b, s: (0, 0)),
            pl.BlockSpec((1, NMASKS), lambda b, s: (0, 0)),
        ],
        out_specs=pl.BlockSpec((1, SB, NMASKS), lambda b, s: (b, s, 0)),
        out_shape=jax.ShapeDtypeStruct((B, SEEDS, NMASKS), jnp.float32),
        scratch_shapes=[pltpu.VMEM((N, FC), jnp.float32)],
        compiler_params=pltpu.CompilerParams(
            dimension_semantics=("arbitrary", "arbitrary")),
    )(pos, masks, posT, seedinfo, W1, b1.reshape(1, D), W2,
      b2.reshape(1, D), Wg, bg.reshape(1, D), Wc1, bc1.reshape(1, D // 2),
      Wc2, bc2.reshape(1, NMASKS))


def _add_body(lg_ref, sm_ref, out_ref):
    out_ref[...] = lg_ref[...] + sm_ref[:, :, 0:NMASKS]


def _addmasks(logits0, smasks):
    return pl.pallas_call(
        _add_body,
        out_shape=jax.ShapeDtypeStruct((B, SEEDS, NMASKS), jnp.float32),
    )(logits0, smasks)


@jax.jit
def kernel(pos, masks, W1, b1, W2, b2, Wg, bg, Wc1, bc1, Wc2, bc2):
    pos = pos.astype(jnp.float32)
    masks = masks.astype(jnp.float32)
    posT = jnp.transpose(pos, (0, 2, 1))
    seedinfo = _fps(jnp.transpose(pos, (2, 0, 1)))
    table = jnp.pad(masks.reshape(B * N, NMASKS),
                    ((0, 0), (0, MPAD - NMASKS)))
    smasks = _sc_gather(seedinfo[:, 0, :].reshape(B * SEEDS), table)
    logits0 = _group(pos, masks, posT, seedinfo, W1, b1, W2, b2,
                     Wg, bg, Wc1, bc1, Wc2, bc2)
    return _addmasks(logits0, smasks.reshape(B, SEEDS, MPAD))


# speedup vs baseline: 1.0467x; 1.0107x over previous
import functools

import jax
import jax.numpy as jnp
import numpy as np
from jax import lax
from jax.experimental import pallas as pl
from jax.experimental.pallas import tpu as pltpu
from jax.experimental.pallas import tpu_sc as plsc

B, N, NMASKS, SEEDS, K, D = 8, 8192, 10, 256, 64, 64
R = 0.05
SB = 256
NSB = SEEDS // SB
FC = 128
MPAD = 16
BITS_3F = int(np.float32(3.0).view(np.int32))
I32_MAX = np.int32(2**31 - 1)


def _radius_d2_bound():
    r = np.float32(R)
    t = r * r
    up = np.float32(np.inf)
    dn = np.float32(-np.inf)
    while np.sqrt(np.nextafter(t, up)) <= r:
        t = np.nextafter(t, up)
    while np.sqrt(t) > r:
        t = np.nextafter(t, dn)
    return float(t)


R2BOUND = _radius_d2_bound()


def _fps_body(post_ref, out_ref):
    px = post_ref[0]
    py = post_ref[1]
    pz = post_ref[2]
    lane = lax.broadcasted_iota(jnp.int32, (B, N), 1).astype(jnp.float32)
    slane = lax.broadcasted_iota(jnp.int32, (B, SEEDS), 1)

    lx0 = px[:, 0:1]
    ly0 = py[:, 0:1]
    lz0 = pz[:, 0:1]
    zeros_s = jnp.zeros((B, SEEDS), jnp.float32)
    w0 = slane == 0
    sxs = jnp.where(w0, lx0, zeros_s)
    sys_ = jnp.where(w0, ly0, zeros_s)
    szs = jnp.where(w0, lz0, zeros_s)
    idxs = zeros_s
    dists = jnp.full((B, N), 1e10, jnp.float32)

    off = jnp.float32(8388608.0)
    laneoff = lane + off

    def body(i, carry):
        dists, idxs, sxs, sys_, szs, lx, ly, lz = carry
        dx = px - lx
        dy = py - ly
        dz = pz - lz
        d = (dx * dx + dy * dy) + dz * dz
        dists = jnp.minimum(dists, d)
        m = jnp.max(dists, axis=1, keepdims=True)
        hotf = (dists == m).astype(jnp.float32)
        si = jnp.sum(hotf * laneoff, axis=1, keepdims=True)
        ex = jnp.sum(hotf * px, axis=1, keepdims=True)
        ey = jnp.sum(hotf * py, axis=1, keepdims=True)
        ez = jnp.sum(hotf * pz, axis=1, keepdims=True)

        def tie_path(_):
            hot = dists == m
            nxt = jnp.min(jnp.where(hot, lane, 1e9), axis=1, keepdims=True)
            h2 = lane == nxt
            tx = jnp.sum(jnp.where(h2, px, 0.0), axis=1, keepdims=True)
            ty = jnp.sum(jnp.where(h2, py, 0.0), axis=1, keepdims=True)
            tz = jnp.sum(jnp.where(h2, pz, 0.0), axis=1, keepdims=True)
            return nxt, tx, ty, tz

        def fast_path(_):
            return si - off, ex, ey, ez

        nxt, lx, ly, lz = lax.cond(jnp.max(si) >= 2.0 * off, tie_path,
                                   fast_path, 0)
        w = slane == i
        idxs = jnp.where(w, nxt, idxs)
        sxs = jnp.where(w, lx, sxs)
        sys_ = jnp.where(w, ly, sys_)
        szs = jnp.where(w, lz, szs)
        return dists, idxs, sxs, sys_, szs, lx, ly, lz

    carry = (dists, idxs, sxs, sys_, szs, lx0, ly0, lz0)
    _, idxs, sxs, sys_, szs, _, _, _ = lax.fori_loop(1, SEEDS, body, carry, unroll=15)
    out_ref[:, 0:1, :] = idxs[:, None, :]
    out_ref[:, 1:2, :] = sxs[:, None, :]
    out_ref[:, 2:3, :] = sys_[:, None, :]
    out_ref[:, 3:4, :] = szs[:, None, :]


def _fps(posT):
    return pl.pallas_call(
        _fps_body,
        out_shape=jax.ShapeDtypeStruct((B, 4, SEEDS), jnp.float32),
    )(posT)


_NC, _NS = 2, 16
_NW = _NC * _NS
_RPW = (B * SEEDS) // _NW


def _sc_gather_body(idxf_hbm, table_hbm, out_hbm, idxf_v, idx_v, rows_v, sem):
    wid = lax.axis_index("s") * _NC + lax.axis_index("c")
    base = wid * _RPW
    boff = (base // SEEDS) * N
    pltpu.sync_copy(idxf_hbm.at[pl.ds(base, _RPW)], idxf_v)
    for j in range(_RPW // 16):
        idx_v[pl.ds(j * 16, 16)] = idxf_v[pl.ds(j * 16, 16)].astype(jnp.int32) + boff
    pltpu.async_copy(table_hbm.at[idx_v], rows_v, sem).wait()
    pltpu.sync_copy(rows_v, out_hbm.at[pl.ds(base, _RPW)])


def _sc_gather(idxf, table):
    mesh = plsc.VectorSubcoreMesh(core_axis_name="c", subcore_axis_name="s")
    fn = functools.partial(
        pl.kernel,
        mesh=mesh,
        out_type=jax.ShapeDtypeStruct((B * SEEDS, MPAD), jnp.float32),
        scratch_types=[
            pltpu.VMEM((_RPW,), jnp.float32),
            pltpu.VMEM((_RPW,), jnp.int32),
            pltpu.VMEM((_RPW, MPAD), jnp.float32),
            pltpu.SemaphoreType.DMA,
        ],
        compiler_params=pltpu.CompilerParams(use_tc_tiling_on_sc=False),
    )(_sc_gather_body)
    return fn(idxf, table)


def _group_body(pos_ref, masks_ref, post_ref, seed_ref, smask_ref,
                w1_ref, b1_ref, w2_ref, b2_ref, wg_ref, bg_ref,
                wc1_ref, bc1_ref, wc2_ref, bc2_ref, out_ref, f_scr):
    sb = pl.program_id(1)

    @pl.when(sb == 0)
    def _build_features():
        p = pos_ref[0]
        h = jnp.maximum(
            jnp.dot(p, w1_ref[...], preferred_element_type=jnp.float32)
            + b1_ref[...], 0.0)
        x = jnp.maximum(
            jnp.dot(h, w2_ref[...], preferred_element_type=jnp.float32)
            + b2_ref[...], 0.0)
        ones = jnp.ones((N, 1), jnp.float32)
        zeros = jnp.zeros((N, FC - D - 1), jnp.float32)
        f_scr[...] = jnp.concatenate([x, ones, zeros], axis=1)

    info = seed_ref[0, :, pl.ds(sb * SB, SB)]
    infoT = jnp.swapaxes(info, 0, 1)
    idxc = infoT[:, 0:1]
    sx = infoT[:, 1:2]
    sy = infoT[:, 2:3]
    sz = infoT[:, 3:4]

    px = post_ref[0, 0:1, :]
    py = post_ref[0, 1:2, :]
    pz = post_ref[0, 2:3, :]
    dx = sx - px
    dy = sy - py
    dz = sz - pz
    d2 = (dx * dx + dy * dy) + dz * dz
    maskR = (d2 <= R2BOUND).astype(jnp.float32)

    lanei = lax.broadcasted_iota(jnp.int32, (SB, N), 1)
    maskH = (lanei == idxc.astype(jnp.int32)).astype(jnp.float32)

    f = f_scr[...]
    aggR = jnp.dot(maskR, f, preferred_element_type=jnp.float32)
    aggH = jnp.dot(maskH, f, preferred_element_type=jnp.float32)

    kf = jnp.float32(K)
    cntR = aggR[:, D:D + 1]

    def refine(_):
        d2b = lax.bitcast_convert_type(d2, jnp.int32)
        lo = jnp.zeros((SB, 1), jnp.int32)
        hi = jnp.full((SB, 1), BITS_3F, jnp.int32)

        def bs(_, lh):
            lo, hi = lh
            mid = (lo + hi) // 2
            c = jnp.sum((d2b <= mid).astype(jnp.float32), axis=1, keepdims=True)
            ge = c >= kf
            return jnp.where(ge, lo, mid + 1), jnp.where(ge, mid, hi)

        lo, hi = lax.fori_loop(0, 31, bs, (lo, hi))
        tau = jnp.where(cntR > kf, hi, I32_MAX)
        maskN = jnp.minimum(maskR, (d2b <= tau).astype(jnp.float32))
        return jnp.dot(maskN, f, preferred_element_type=jnp.float32)

    aggN = lax.cond(jnp.max(cntR) > kf, refine, lambda _: aggR, 0)

    nn_sum = aggN[:, 0:D]
    cnt = aggN[:, D:D + 1]
    seed_x = aggH[:, 0:D]

    nn_avg = nn_sum / jnp.maximum(cnt, 1e-9)
    disc = seed_x - nn_avg
    gin = jnp.concatenate([seed_x, nn_avg], axis=1)
    grp = jnp.maximum(
        jnp.dot(gin, wg_ref[...], preferred_element_type=jnp.float32)
        + bg_ref[...], 0.0) + disc
    h = jnp.maximum(
        jnp.dot(grp, wc1_ref[...], preferred_element_type=jnp.float32)
        + bc1_ref[...], 0.0)
    out_ref[0] = (jnp.dot(h, wc2_ref[...], preferred_element_type=jnp.float32)
                  + bc2_ref[...] + smask_ref[0][:, 0:NMASKS])


def _group(pos, masks, posT, seedinfo, smasks, W1, b1, W2, b2,
           Wg, bg, Wc1, bc1, Wc2, bc2):
    return pl.pallas_call(
        _group_body,
        grid=(B, NSB),
        in_specs=[
            pl.BlockSpec((1, N, 3), lambda b, s: (b, 0, 0)),
            pl.BlockSpec((1, N, NMASKS), lambda b, s: (b, 0, 0)),
            pl.BlockSpec((1, 3, N), lambda b, s: (b, 0, 0)),
            pl.BlockSpec((1, 4, SEEDS), lambda b, s: (b, 0, 0)),
            pl.BlockSpec((1, SB, MPAD), lambda b, s: (b, 0, 0)),
            pl.BlockSpec((3, D), lambda b, s: (0, 0)),
            pl.BlockSpec((1, D), lambda b, s: (0, 0)),
            pl.BlockSpec((D, D), lambda b, s: (0, 0)),
            pl.BlockSpec((1, D), lambda b, s: (0, 0)),
            pl.BlockSpec((2 * D, D), lambda b, s: (0, 0)),
            pl.BlockSpec((1, D), lambda b, s: (0, 0)),
            pl.BlockSpec((D, D // 2), lambda b, s: (0, 0)),
            pl.BlockSpec((1, D // 2), lambda b, s: (0, 0)),
            pl.BlockSpec((D // 2, NMASKS), lambda b, s: (0, 0)),
            pl.BlockSpec((1, NMASKS), lambda b, s: (0, 0)),
        ],
        out_specs=pl.BlockSpec((1, SB, NMASKS), lambda b, s: (b, s, 0)),
        out_shape=jax.ShapeDtypeStruct((B, SEEDS, NMASKS), jnp.float32),
        scratch_shapes=[pltpu.VMEM((N, FC), jnp.float32)],
        compiler_params=pltpu.CompilerParams(
            dimension_semantics=("arbitrary", "arbitrary")),
    )(pos, masks, posT, seedinfo, smasks, W1, b1.reshape(1, D), W2,
      b2.reshape(1, D), Wg, bg.reshape(1, D), Wc1, bc1.reshape(1, D // 2),
      Wc2, bc2.reshape(1, NMASKS))


@jax.jit
def kernel(pos, masks, W1, b1, W2, b2, Wg, bg, Wc1, bc1, Wc2, bc2):
    pos = pos.astype(jnp.float32)
    masks = masks.astype(jnp.float32)
    posT = jnp.transpose(pos, (0, 2, 1))
    seedinfo = _fps(jnp.transpose(pos, (2, 0, 1)))
    table = jnp.pad(masks.reshape(B * N, NMASKS),
                    ((0, 0), (0, MPAD - NMASKS)))
    smasks = _sc_gather(seedinfo[:, 0, :].reshape(B * SEEDS), table)
    return _group(pos, masks, posT, seedinfo,
                  smasks.reshape(B, SEEDS, MPAD), W1, b1, W2, b2,
                  Wg, bg, Wc1, bc1, Wc2, bc2)
